# baseline (speedup 1.0000x reference)
import functools

import jax
import jax.numpy as jnp
from jax.experimental import pallas as pl
from jax.experimental.pallas import tpu as pltpu

N = 10000
H = 512
MB = 1000


def _mm_kernel(x_ref, wt_ref, b_ref, o_ref, *, relu):
    y = jnp.dot(x_ref[...], wt_ref[...]) + b_ref[...][None, :]
    if relu:
        y = jnp.maximum(y, 0.0)
    o_ref[...] = y


def _mm(x, wt, b, relu=False):
    n, k = x.shape
    m = wt.shape[1]
    return pl.pallas_call(
        functools.partial(_mm_kernel, relu=relu),
        grid=(n // MB,),
        in_specs=[
            pl.BlockSpec((MB, k), lambda i: (i, 0)),
            pl.BlockSpec((k, m), lambda i: (0, 0)),
            pl.BlockSpec((m,), lambda i: (0,)),
        ],
        out_specs=pl.BlockSpec((MB, m), lambda i: (i, 0)),
        out_shape=jax.ShapeDtypeStruct((n, m), jnp.float32),
    )(x, wt, b)


def kernel(x, edge_index, Win, b_in, Wk, bk, Wq, bq, Wv, bv, Ws, bs,
           gamma, beta, Wpost, bpost):
    src = edge_index[0]
    dst = edge_index[1]
    bn_scale = 1.0 / jnp.sqrt(1.0 + 1e-5)
    L = Wk.shape[0]

    h = _mm(x, Win.T, b_in, relu=True)
    for l in range(L):
        wt = jnp.concatenate([Wk[l], Wq[l], Wv[l], Ws[l]], axis=0).T
        bfused = jnp.concatenate([bk[l], bq[l], bv[l], bs[l]])
        y = _mm(h, wt, bfused)
        k = y[:, 0:H]
        q = y[:, H:2 * H]
        v = y[:, 2 * H:3 * H]
        s = y[:, 3 * H:4 * H]
        msg = jax.nn.sigmoid(k[dst] + q[src]) * v[src]
        aggr = jnp.zeros_like(h).at[dst].add(msg)
        conv = aggr + s
        bnv = conv * bn_scale * gamma[l] + beta[l]
        h = h + jax.nn.relu(bnv)
    return _mm(h, Wpost.T, bpost)


# speedup vs baseline: 2.2960x; 2.2960x over previous
import functools

import jax
import jax.numpy as jnp
from jax import lax
from jax.experimental import pallas as pl
from jax.experimental.pallas import tpu as pltpu
from jax.experimental.pallas import tpu_sc as plsc

N = 10000
H = 512
E = 160000
MB = 1000
CW = 128
NCHUNK = H // CW
NS = 16
B = 80
EPS = E // NS
NB = EPS // B
ROWS_A = 640
ZR = 80
BN_SCALE = 1.0 / (1.0 + 1e-5) ** 0.5


def _mm_kernel(x_ref, wt_ref, b_ref, o_ref, *, relu):
    y = jnp.dot(x_ref[...], wt_ref[...]) + b_ref[...][None, :]
    if relu:
        y = jnp.maximum(y, 0.0)
    o_ref[...] = y


def _mm(x, wt, b, relu=False):
    n, k = x.shape
    m = wt.shape[1]
    return pl.pallas_call(
        functools.partial(_mm_kernel, relu=relu),
        grid=(n // MB,),
        in_specs=[
            pl.BlockSpec((MB, k), lambda i: (i, 0)),
            pl.BlockSpec((k, m), lambda i: (0, 0)),
            pl.BlockSpec((m,), lambda i: (0,)),
        ],
        out_specs=pl.BlockSpec((MB, m), lambda i: (i, 0)),
        out_shape=jax.ShapeDtypeStruct((n, m), jnp.float32),
    )(x, wt, b)


def _mmc_kernel(x_ref, wt_ref, b_ref, o_ref):
    o_ref[...] = jnp.dot(x_ref[...], wt_ref[...]) + b_ref[0, 0, :][None, :]


def _mmc(x, wt, b16):
    return pl.pallas_call(
        _mmc_kernel,
        grid=(N // MB, 16),
        in_specs=[
            pl.BlockSpec((MB, H), lambda m, n: (m, 0)),
            pl.BlockSpec((H, CW), lambda m, n: (0, n)),
            pl.BlockSpec((1, 1, CW), lambda m, n: (n, 0, 0)),
        ],
        out_specs=pl.BlockSpec((MB, CW), lambda m, n: (n * (N // MB) + m, 0)),
        out_shape=jax.ShapeDtypeStruct((16 * N, CW), jnp.float32),
    )(x, wt, b16)


def _update_kernel(h_ref, a0, a1, a2, a3, s0, s1, s2, s3, g_ref, be_ref, o_ref):
    aggr = jnp.concatenate([a0[...], a1[...], a2[...], a3[...]], axis=1)
    sm = jnp.concatenate([s0[...], s1[...], s2[...], s3[...]], axis=1)
    bnv = (aggr + sm) * (BN_SCALE * g_ref[...][None, :]) + be_ref[...][None, :]
    o_ref[...] = h_ref[...] + jnp.maximum(bnv, 0.0)


def _update(h, aggr, yt, gam, bet):
    nblk = N // MB
    aspec = [pl.BlockSpec((MB, CW), (lambda m, g=g: (g * nblk + m, 0)))
             for g in range(NCHUNK)]
    sspec = [pl.BlockSpec((MB, CW), (lambda m, g=g: ((12 + g) * nblk + m, 0)))
             for g in range(NCHUNK)]
    return pl.pallas_call(
        _update_kernel,
        grid=(nblk,),
        in_specs=[pl.BlockSpec((MB, H), lambda m: (m, 0))] + aspec + sspec + [
            pl.BlockSpec((H,), lambda m: (0,)),
            pl.BlockSpec((H,), lambda m: (0,)),
        ],
        out_specs=pl.BlockSpec((MB, H), lambda m: (m, 0)),
        out_shape=jax.ShapeDtypeStruct((N, H), jnp.float32),
    )(h, aggr, aggr, aggr, aggr, yt, yt, yt, yt, gam, bet)


def _edge_body(yt, srcd, dstd, out,
               didx, sidx, kidx, qidx, vidx, tbuf, vbuf, zbuf, acc,
               sem0, sem1):
    c = lax.axis_index("c")
    s = lax.axis_index("s")

    def zrow(r, _):
        for i in range(CW // 16):
            zbuf[r, pl.ds(i * 16, 16)] = jnp.zeros((16,), jnp.float32)
        return 0
    lax.fori_loop(0, ZR, zrow, 0)

    start = s * ROWS_A
    ncopies = jnp.where(s < NS - 1, ROWS_A // ZR, (N - (NS - 1) * ROWS_A) // ZR)

    for half in range(2):
        g = c * 2 + half
        def zc(j, _):
            pltpu.sync_copy(zbuf, acc.at[pl.ds(start + j * ZR, ZR)])
            return 0
        lax.fori_loop(0, ncopies, zc, 0)
        plsc.subcore_barrier()

        off_k = g * N
        off_q = (4 + g) * N
        off_v = (8 + g) * N

        def batch(b, _):
            e0 = s * EPS + b * B
            pltpu.sync_copy(dstd.at[pl.ds(e0, B)], didx)
            pltpu.sync_copy(srcd.at[pl.ds(e0, B)], sidx)
            for i in range(B // 16):
                sl = pl.ds(i * 16, 16)
                d16 = didx[sl]
                s16 = sidx[sl]
                kidx[sl] = d16 + off_k
                qidx[sl] = s16 + off_q
                vidx[sl] = s16 + off_v
            ck = pltpu.async_copy(yt.at[kidx], tbuf, sem0)
            cv = pltpu.async_copy(yt.at[vidx], vbuf, sem1)
            ck.wait()
            pltpu.async_copy(yt.at[qidx], tbuf, sem0, add=True).wait()
            cv.wait()

            def edge(e, _):
                for i in range(CW // 16):
                    sl = pl.ds(i * 16, 16)
                    t = tbuf[e, sl]
                    vv = vbuf[e, sl]
                    vbuf[e, sl] = vv / (1.0 + jnp.exp(-t))
                return 0
            lax.fori_loop(0, B, edge, 0)

            pltpu.sync_copy(vbuf, acc.at[didx], add=True)
            return 0
        lax.fori_loop(0, NB, batch, 0)

        plsc.subcore_barrier()

        def fc(j, _):
            pltpu.sync_copy(acc.at[pl.ds(start + j * ZR, ZR)],
                            out.at[pl.ds(g * N + start + j * ZR, ZR)])
            return 0
        lax.fori_loop(0, ncopies, fc, 0)
        plsc.subcore_barrier()


_edge_call = pl.kernel(
    _edge_body,
    out_type=jax.ShapeDtypeStruct((NCHUNK * N, CW), jnp.float32),
    mesh=plsc.VectorSubcoreMesh(core_axis_name="c", subcore_axis_name="s",
                                num_cores=2, num_subcores=NS),
    scratch_types=[
        pltpu.VMEM((B,), jnp.int32),
        pltpu.VMEM((B,), jnp.int32),
        pltpu.VMEM((B,), jnp.int32),
        pltpu.VMEM((B,), jnp.int32),
        pltpu.VMEM((B,), jnp.int32),
        pltpu.VMEM((B, CW), jnp.float32),
        pltpu.VMEM((B, CW), jnp.float32),
        pltpu.VMEM((ZR, CW), jnp.float32),
        pltpu.VMEM_SHARED((N, CW), jnp.float32),
        pltpu.SemaphoreType.DMA,
        pltpu.SemaphoreType.DMA,
    ],
)


def kernel(x, edge_index, Win, b_in, Wk, bk, Wq, bq, Wv, bv, Ws, bs,
           gamma, beta, Wpost, bpost):
    src = edge_index[0]
    dst = edge_index[1]
    L = Wk.shape[0]

    h = _mm(x, Win.T, b_in, relu=True)
    for l in range(L):
        wt = jnp.concatenate([Wk[l], Wq[l], Wv[l], Ws[l]], axis=0).T
        bf = jnp.concatenate([bk[l], bq[l], bv[l], bs[l]]).reshape(16, 1, CW)
        yt = _mmc(h, wt, bf)
        aggr = _edge_call(yt, src, dst)
        h = _update(h, aggr, yt, gamma[l], beta[l])
    return _mm(h, Wpost.T, bpost)


# speedup vs baseline: 3.7465x; 1.6317x over previous
import functools

import jax
import jax.numpy as jnp
from jax import lax
from jax.experimental import pallas as pl
from jax.experimental.pallas import tpu as pltpu
from jax.experimental.pallas import tpu_sc as plsc

N = 10000
H = 512
E = 160000
MB = 1000
CW = 128
NCHUNK = H // CW
NS = 16
B = 80
EPS = E // NS
NB = EPS // B
ROWS_A = 640
ZR = 80
ZRZ = 40
BN_SCALE = 1.0 / (1.0 + 1e-5) ** 0.5


def _mm_kernel(x_ref, wt_ref, b_ref, o_ref, *, relu):
    y = jnp.dot(x_ref[...], wt_ref[...]) + b_ref[...][None, :]
    if relu:
        y = jnp.maximum(y, 0.0)
    o_ref[...] = y


def _mm(x, wt, b, relu=False):
    n, k = x.shape
    m = wt.shape[1]
    return pl.pallas_call(
        functools.partial(_mm_kernel, relu=relu),
        grid=(n // MB,),
        in_specs=[
            pl.BlockSpec((MB, k), lambda i: (i, 0)),
            pl.BlockSpec((k, m), lambda i: (0, 0)),
            pl.BlockSpec((m,), lambda i: (0,)),
        ],
        out_specs=pl.BlockSpec((MB, m), lambda i: (i, 0)),
        out_shape=jax.ShapeDtypeStruct((n, m), jnp.float32),
    )(x, wt, b)


def _mmc_kernel(x_ref, wt_ref, b_ref, o_ref):
    o_ref[...] = jnp.dot(x_ref[...], wt_ref[...]) + b_ref[0, 0, :][None, :]


def _mmc(x, wt, b16):
    return pl.pallas_call(
        _mmc_kernel,
        grid=(N // MB, 16),
        in_specs=[
            pl.BlockSpec((MB, H), lambda m, n: (m, 0)),
            pl.BlockSpec((H, CW), lambda m, n: (0, n)),
            pl.BlockSpec((1, 1, CW), lambda m, n: (n, 0, 0)),
        ],
        out_specs=pl.BlockSpec((MB, CW), lambda m, n: (n * (N // MB) + m, 0)),
        out_shape=jax.ShapeDtypeStruct((16 * N, CW), jnp.float32),
    )(x, wt, b16)


def _update_kernel(h_ref, a0, a1, a2, a3, s0, s1, s2, s3, g_ref, be_ref, o_ref):
    aggr = jnp.concatenate([a0[...], a1[...], a2[...], a3[...]], axis=1)
    sm = jnp.concatenate([s0[...], s1[...], s2[...], s3[...]], axis=1)
    bnv = (aggr + sm) * (BN_SCALE * g_ref[...][None, :]) + be_ref[...][None, :]
    o_ref[...] = h_ref[...] + jnp.maximum(bnv, 0.0)


def _update(h, aggr, yt, gam, bet):
    nblk = N // MB
    aspec = [pl.BlockSpec((MB, CW), (lambda m, g=g: (g * nblk + m, 0)))
             for g in range(NCHUNK)]
    sspec = [pl.BlockSpec((MB, CW), (lambda m, g=g: ((12 + g) * nblk + m, 0)))
             for g in range(NCHUNK)]
    return pl.pallas_call(
        _update_kernel,
        grid=(nblk,),
        in_specs=[pl.BlockSpec((MB, H), lambda m: (m, 0))] + aspec + sspec + [
            pl.BlockSpec((H,), lambda m: (0,)),
            pl.BlockSpec((H,), lambda m: (0,)),
        ],
        out_specs=pl.BlockSpec((MB, H), lambda m: (m, 0)),
        out_shape=jax.ShapeDtypeStruct((N, H), jnp.float32),
    )(h, aggr, aggr, aggr, aggr, yt, yt, yt, yt, gam, bet)


def _edge_body(yt, srcd, dstd, out,
               didx0, sidx0, kidx0, qidx0, vidx0, tbuf0, vbuf0,
               didx1, sidx1, kidx1, qidx1, vidx1, tbuf1, vbuf1,
               zbuf, acc,
               semi0, semk0, semq0, semv0,
               semi1, semk1, semq1, semv1):
    c = lax.axis_index("c")
    s = lax.axis_index("s")

    BUFS = [
        (didx0, sidx0, kidx0, qidx0, vidx0, tbuf0, vbuf0,
         semi0, semk0, semq0, semv0),
        (didx1, sidx1, kidx1, qidx1, vidx1, tbuf1, vbuf1,
         semi1, semk1, semq1, semv1),
    ]

    def zrow(r, _):
        for i in range(CW // 16):
            zbuf[r, pl.ds(i * 16, 16)] = jnp.zeros((16,), jnp.float32)
        return 0
    lax.fori_loop(0, ZRZ, zrow, 0)

    start = s * ROWS_A
    rows_mine = jnp.where(s < NS - 1, ROWS_A, N - (NS - 1) * ROWS_A)
    nzero = rows_mine // ZRZ
    ncopies = rows_mine // ZR

    def idx_load_async(b, p):
        didx, sidx = BUFS[p][0], BUFS[p][1]
        semi = BUFS[p][7]
        e0 = s * EPS + b * B
        pltpu.async_copy(dstd.at[pl.ds(e0, B)], didx, semi)
        pltpu.async_copy(srcd.at[pl.ds(e0, B)], sidx, semi)

    def idx_wait(b, p):
        didx, sidx = BUFS[p][0], BUFS[p][1]
        semi = BUFS[p][7]
        e0 = s * EPS + b * B
        pltpu.make_async_copy(dstd.at[pl.ds(e0, B)], didx, semi).wait()
        pltpu.make_async_copy(srcd.at[pl.ds(e0, B)], sidx, semi).wait()

    for half in range(2):
        g = c * 2 + half
        def zc(j, _):
            pltpu.sync_copy(zbuf, acc.at[pl.ds(start + j * ZRZ, ZRZ)])
            return 0
        lax.fori_loop(0, nzero, zc, 0)
        plsc.subcore_barrier()

        off_k = g * N
        off_q = (4 + g) * N
        off_v = (8 + g) * N

        def shift(p):
            didx, sidx, kidx, qidx, vidx = BUFS[p][:5]
            for i in range(B // 16):
                sl = pl.ds(i * 16, 16)
                d16 = didx[sl]
                s16 = sidx[sl]
                kidx[sl] = d16 + off_k
                qidx[sl] = s16 + off_q
                vidx[sl] = s16 + off_v

        def issue_kv(p):
            kidx, _, vidx, tbuf, vbuf = BUFS[p][2:7]
            semk, _, semv = BUFS[p][8:11]
            pltpu.async_copy(yt.at[kidx], tbuf, semk)
            pltpu.async_copy(yt.at[vidx], vbuf, semv)

        def step(b, p):
            didx = BUFS[p][0]
            kidx, qidx, vidx, tbuf, vbuf = BUFS[p][2:7]
            semk, semq, semv = BUFS[p][8:11]
            p1 = 1 - p
            pltpu.make_async_copy(yt.at[kidx], tbuf, semk).wait()
            pltpu.async_copy(yt.at[qidx], tbuf, semq, add=True)

            @pl.when(b + 1 < NB)
            def _():
                idx_wait(b + 1, p1)
                shift(p1)
                issue_kv(p1)

            pltpu.make_async_copy(yt.at[qidx], tbuf, semq).wait()
            pltpu.make_async_copy(yt.at[vidx], vbuf, semv).wait()

            def edge(e, _):
                for i in range(CW // 16):
                    sl = pl.ds(i * 16, 16)
                    t = tbuf[e, sl]
                    vv = vbuf[e, sl]
                    vbuf[e, sl] = vv / (1.0 + jnp.exp(-t))
                return 0
            lax.fori_loop(0, B, edge, 0)

            pltpu.sync_copy(vbuf, acc.at[didx], add=True)

            @pl.when(b + 2 < NB)
            def _():
                idx_load_async(b + 2, p)

        idx_load_async(0, 0)
        idx_wait(0, 0)
        shift(0)
        issue_kv(0)
        idx_load_async(1, 1)
        step(0, 0)

        def pair(i, _):
            step(2 * i + 1, 1)
            step(2 * i + 2, 0)
            return 0
        lax.fori_loop(0, (NB - 1) // 2, pair, 0)

        plsc.subcore_barrier()

        def fc(j, _):
            pltpu.sync_copy(acc.at[pl.ds(start + j * ZR, ZR)],
                            out.at[pl.ds(g * N + start + j * ZR, ZR)])
            return 0
        lax.fori_loop(0, ncopies, fc, 0)
        plsc.subcore_barrier()


_edge_call = pl.kernel(
    _edge_body,
    out_type=jax.ShapeDtypeStruct((NCHUNK * N, CW), jnp.float32),
    mesh=plsc.VectorSubcoreMesh(core_axis_name="c", subcore_axis_name="s",
                                num_cores=2, num_subcores=NS),
    scratch_types=(
        [pltpu.VMEM((B,), jnp.int32)] * 5
        + [pltpu.VMEM((B, CW), jnp.float32)] * 2
        + [pltpu.VMEM((B,), jnp.int32)] * 5
        + [pltpu.VMEM((B, CW), jnp.float32)] * 2
        + [pltpu.VMEM((ZRZ, CW), jnp.float32)]
        + [pltpu.VMEM_SHARED((N, CW), jnp.float32)]
        + [pltpu.SemaphoreType.DMA] * 8
    ),
)


def kernel(x, edge_index, Win, b_in, Wk, bk, Wq, bq, Wv, bv, Ws, bs,
           gamma, beta, Wpost, bpost):
    src = edge_index[0]
    dst = edge_index[1]
    L = Wk.shape[0]

    h = _mm(x, Win.T, b_in, relu=True)
    for l in range(L):
        wt = jnp.concatenate([Wk[l], Wq[l], Wv[l], Ws[l]], axis=0).T
        bf = jnp.concatenate([bk[l], bq[l], bv[l], bs[l]]).reshape(16, 1, CW)
        yt = _mmc(h, wt, bf)
        aggr = _edge_call(yt, src, dst)
        h = _update(h, aggr, yt, gamma[l], beta[l])
    return _mm(h, Wpost.T, bpost)


# speedup vs baseline: 4.0086x; 1.0699x over previous
import functools

import jax
import jax.numpy as jnp
from jax import lax
from jax.experimental import pallas as pl
from jax.experimental.pallas import tpu as pltpu
from jax.experimental.pallas import tpu_sc as plsc

N = 10000
H = 512
E = 160000
MB = 1000
CW = 128
NCHUNK = H // CW
NS = 16
B = 80
EPS = E // NS
NB = EPS // B
ROWS_A = 640
ZR = 80
ZRZ = 40
BN_SCALE = 1.0 / (1.0 + 1e-5) ** 0.5


def _mm_kernel(x_ref, wt_ref, b_ref, o_ref, *, relu):
    y = jnp.dot(x_ref[...], wt_ref[...]) + b_ref[...][None, :]
    if relu:
        y = jnp.maximum(y, 0.0)
    o_ref[...] = y


def _mm(x, wt, b, relu=False):
    n, k = x.shape
    m = wt.shape[1]
    return pl.pallas_call(
        functools.partial(_mm_kernel, relu=relu),
        grid=(n // MB,),
        in_specs=[
            pl.BlockSpec((MB, k), lambda i: (i, 0)),
            pl.BlockSpec((k, m), lambda i: (0, 0)),
            pl.BlockSpec((m,), lambda i: (0,)),
        ],
        out_specs=pl.BlockSpec((MB, m), lambda i: (i, 0)),
        out_shape=jax.ShapeDtypeStruct((n, m), jnp.float32),
    )(x, wt, b)


def _mmc_kernel(x_ref, wt_ref, b_ref, o_ref):
    o_ref[...] = jnp.dot(x_ref[...], wt_ref[...]) + b_ref[0, 0, :][None, :]


def _mmc(x, wt, b16):
    return pl.pallas_call(
        _mmc_kernel,
        grid=(N // MB, 16),
        in_specs=[
            pl.BlockSpec((MB, H), lambda m, n: (m, 0)),
            pl.BlockSpec((H, CW), lambda m, n: (0, n)),
            pl.BlockSpec((1, 1, CW), lambda m, n: (n, 0, 0)),
        ],
        out_specs=pl.BlockSpec((MB, CW), lambda m, n: (n * (N // MB) + m, 0)),
        out_shape=jax.ShapeDtypeStruct((16 * N, CW), jnp.float32),
    )(x, wt, b16)


def _update_kernel(h_ref, a0, a1, a2, a3, s0, s1, s2, s3, g_ref, be_ref, o_ref):
    aggr = jnp.concatenate([a0[...], a1[...], a2[...], a3[...]], axis=1)
    sm = jnp.concatenate([s0[...], s1[...], s2[...], s3[...]], axis=1)
    bnv = (aggr + sm) * (BN_SCALE * g_ref[...][None, :]) + be_ref[...][None, :]
    o_ref[...] = h_ref[...] + jnp.maximum(bnv, 0.0)


def _update(h, aggr, yt, gam, bet):
    nblk = N // MB
    aspec = [pl.BlockSpec((MB, CW), (lambda m, g=g: (g * nblk + m, 0)))
             for g in range(NCHUNK)]
    sspec = [pl.BlockSpec((MB, CW), (lambda m, g=g: ((12 + g) * nblk + m, 0)))
             for g in range(NCHUNK)]
    return pl.pallas_call(
        _update_kernel,
        grid=(nblk,),
        in_specs=[pl.BlockSpec((MB, H), lambda m: (m, 0))] + aspec + sspec + [
            pl.BlockSpec((H,), lambda m: (0,)),
            pl.BlockSpec((H,), lambda m: (0,)),
        ],
        out_specs=pl.BlockSpec((MB, H), lambda m: (m, 0)),
        out_shape=jax.ShapeDtypeStruct((N, H), jnp.float32),
    )(h, aggr, aggr, aggr, aggr, yt, yt, yt, yt, gam, bet)


def _edge_body(yt, srcd, dstd, out,
               didx0, sidx0, kidx0, qidx0, vidx0, tbuf0, vbuf0, sdidx0,
               didx1, sidx1, kidx1, qidx1, vidx1, tbuf1, vbuf1, sdidx1,
               zbuf, acc,
               semi0, semk0, semq0, semv0, semsc0,
               semi1, semk1, semq1, semv1, semsc1):
    c = lax.axis_index("c")
    s = lax.axis_index("s")

    BUFS = [
        (didx0, sidx0, kidx0, qidx0, vidx0, tbuf0, vbuf0,
         semi0, semk0, semq0, semv0, sdidx0, semsc0),
        (didx1, sidx1, kidx1, qidx1, vidx1, tbuf1, vbuf1,
         semi1, semk1, semq1, semv1, sdidx1, semsc1),
    ]

    def zrow(r, _):
        for i in range(CW // 16):
            zbuf[r, pl.ds(i * 16, 16)] = jnp.zeros((16,), jnp.float32)
        return 0
    lax.fori_loop(0, ZRZ, zrow, 0)

    start = s * ROWS_A
    rows_mine = jnp.where(s < NS - 1, ROWS_A, N - (NS - 1) * ROWS_A)
    nzero = rows_mine // ZRZ
    ncopies = rows_mine // ZR

    def idx_load_async(b, p):
        didx, sidx = BUFS[p][0], BUFS[p][1]
        semi = BUFS[p][7]
        e0 = s * EPS + b * B
        pltpu.async_copy(dstd.at[pl.ds(e0, B)], didx, semi)
        pltpu.async_copy(srcd.at[pl.ds(e0, B)], sidx, semi)

    def idx_wait(b, p):
        didx, sidx = BUFS[p][0], BUFS[p][1]
        semi = BUFS[p][7]
        e0 = s * EPS + b * B
        pltpu.make_async_copy(dstd.at[pl.ds(e0, B)], didx, semi).wait()
        pltpu.make_async_copy(srcd.at[pl.ds(e0, B)], sidx, semi).wait()

    for half in range(2):
        g = c * 2 + half
        def zc(j, _):
            pltpu.sync_copy(zbuf, acc.at[pl.ds(start + j * ZRZ, ZRZ)])
            return 0
        lax.fori_loop(0, nzero, zc, 0)
        plsc.subcore_barrier()

        off_k = g * N
        off_q = (4 + g) * N
        off_v = (8 + g) * N

        def shift(p):
            didx, sidx, kidx, qidx, vidx = BUFS[p][:5]
            for i in range(B // 16):
                sl = pl.ds(i * 16, 16)
                d16 = didx[sl]
                s16 = sidx[sl]
                kidx[sl] = d16 + off_k
                qidx[sl] = s16 + off_q
                vidx[sl] = s16 + off_v

        def issue_kv(p):
            kidx, _, vidx, tbuf, vbuf = BUFS[p][2:7]
            semk, _, semv = BUFS[p][8:11]
            pltpu.async_copy(yt.at[kidx], tbuf, semk)
            pltpu.async_copy(yt.at[vidx], vbuf, semv)

        def step(b, p, first=False):
            didx = BUFS[p][0]
            kidx, qidx, vidx, tbuf, vbuf = BUFS[p][2:7]
            semk, semq, semv = BUFS[p][8:11]
            sdidx, semsc = BUFS[p][11:13]
            p1 = 1 - p
            tbuf1, vbuf1 = BUFS[p1][5:7]
            kidx1, qidx1 = BUFS[p1][2:4]
            semk1, semq1 = BUFS[p1][8:10]
            sdidx1, semsc1 = BUFS[p1][11:13]

            @pl.when(b + 1 < NB)
            def _():
                idx_wait(b + 1, p1)
                shift(p1)

            if not first:
                pltpu.make_async_copy(vbuf1, acc.at[sdidx1], semsc1).wait()

            @pl.when(b + 1 < NB)
            def _():
                issue_kv(p1)

            pltpu.make_async_copy(yt.at[qidx], tbuf, semq).wait()
            pltpu.make_async_copy(yt.at[vidx], vbuf, semv).wait()

            def edge(e, _):
                for i in range(CW // 16):
                    sl = pl.ds(i * 16, 16)
                    t = tbuf[e, sl]
                    vv = vbuf[e, sl]
                    vbuf[e, sl] = vv / (1.0 + jnp.exp(-t))
                return 0
            lax.fori_loop(0, B, edge, 0)

            for i in range(B // 16):
                sl = pl.ds(i * 16, 16)
                sdidx[sl] = didx[sl]
            pltpu.async_copy(vbuf, acc.at[sdidx], semsc, add=True)

            @pl.when(b + 2 < NB)
            def _():
                idx_load_async(b + 2, p)

            @pl.when(b + 1 < NB)
            def _():
                pltpu.make_async_copy(yt.at[kidx1], tbuf1, semk1).wait()
                pltpu.async_copy(yt.at[qidx1], tbuf1, semq1, add=True)

        idx_load_async(0, 0)
        idx_wait(0, 0)
        shift(0)
        issue_kv(0)
        pltpu.make_async_copy(yt.at[BUFS[0][2]], BUFS[0][5], BUFS[0][8]).wait()
        pltpu.async_copy(yt.at[BUFS[0][3]], BUFS[0][5], BUFS[0][9], add=True)
        idx_load_async(1, 1)
        step(0, 0, first=True)

        def pair(i, _):
            step(2 * i + 1, 1)
            step(2 * i + 2, 0)
            return 0
        lax.fori_loop(0, (NB - 1) // 2, pair, 0)

        pltpu.make_async_copy(BUFS[0][6], acc.at[BUFS[0][11]], BUFS[0][12]).wait()

        plsc.subcore_barrier()

        def fc(j, _):
            pltpu.sync_copy(acc.at[pl.ds(start + j * ZR, ZR)],
                            out.at[pl.ds(g * N + start + j * ZR, ZR)])
            return 0
        lax.fori_loop(0, ncopies, fc, 0)
        plsc.subcore_barrier()


_edge_call = pl.kernel(
    _edge_body,
    out_type=jax.ShapeDtypeStruct((NCHUNK * N, CW), jnp.float32),
    mesh=plsc.VectorSubcoreMesh(core_axis_name="c", subcore_axis_name="s",
                                num_cores=2, num_subcores=NS),
    scratch_types=(
        [pltpu.VMEM((B,), jnp.int32)] * 5
        + [pltpu.VMEM((B, CW), jnp.float32)] * 2
        + [pltpu.VMEM((B,), jnp.int32)]
        + [pltpu.VMEM((B,), jnp.int32)] * 5
        + [pltpu.VMEM((B, CW), jnp.float32)] * 2
        + [pltpu.VMEM((B,), jnp.int32)]
        + [pltpu.VMEM((ZRZ, CW), jnp.float32)]
        + [pltpu.VMEM_SHARED((N, CW), jnp.float32)]
        + [pltpu.SemaphoreType.DMA] * 10
    ),
)


def kernel(x, edge_index, Win, b_in, Wk, bk, Wq, bq, Wv, bv, Ws, bs,
           gamma, beta, Wpost, bpost):
    src = edge_index[0]
    dst = edge_index[1]
    L = Wk.shape[0]

    h = _mm(x, Win.T, b_in, relu=True)
    for l in range(L):
        wt = jnp.concatenate([Wk[l], Wq[l], Wv[l], Ws[l]], axis=0).T
        bf = jnp.concatenate([bk[l], bq[l], bv[l], bs[l]]).reshape(16, 1, CW)
        yt = _mmc(h, wt, bf)
        aggr = _edge_call(yt, src, dst)
        h = _update(h, aggr, yt, gamma[l], beta[l])
    return _mm(h, Wpost.T, bpost)


# speedup vs baseline: 4.0094x; 1.0002x over previous
import functools

import jax
import jax.numpy as jnp
from jax import lax
from jax.experimental import pallas as pl
from jax.experimental.pallas import tpu as pltpu
from jax.experimental.pallas import tpu_sc as plsc

N = 10000
H = 512
E = 160000
MB = 1000
CW = 128
NCHUNK = H // CW
NS = 16
B = 80
EPS = E // NS
NB = EPS // B
ROWS_A = 640
ZR = 80
ZRZ = 40
BN_SCALE = 1.0 / (1.0 + 1e-5) ** 0.5


def _mm_kernel(x_ref, wt_ref, b_ref, o_ref, *, relu):
    y = jnp.dot(x_ref[...].astype(jnp.bfloat16), wt_ref[...].astype(jnp.bfloat16),
                preferred_element_type=jnp.float32) + b_ref[...][None, :]
    if relu:
        y = jnp.maximum(y, 0.0)
    o_ref[...] = y


def _mm(x, wt, b, relu=False):
    n, k = x.shape
    m = wt.shape[1]
    return pl.pallas_call(
        functools.partial(_mm_kernel, relu=relu),
        grid=(n // MB,),
        in_specs=[
            pl.BlockSpec((MB, k), lambda i: (i, 0)),
            pl.BlockSpec((k, m), lambda i: (0, 0)),
            pl.BlockSpec((m,), lambda i: (0,)),
        ],
        out_specs=pl.BlockSpec((MB, m), lambda i: (i, 0)),
        out_shape=jax.ShapeDtypeStruct((n, m), jnp.float32),
    )(x, wt, b)


def _mmc_kernel(x_ref, wt_ref, b_ref, o_ref):
    o_ref[...] = jnp.dot(x_ref[...].astype(jnp.bfloat16),
                         wt_ref[...].astype(jnp.bfloat16),
                         preferred_element_type=jnp.float32) + b_ref[0, 0, :][None, :]


def _mmc(x, wt, b16):
    return pl.pallas_call(
        _mmc_kernel,
        grid=(N // MB, 16),
        in_specs=[
            pl.BlockSpec((MB, H), lambda m, n: (m, 0)),
            pl.BlockSpec((H, CW), lambda m, n: (0, n)),
            pl.BlockSpec((1, 1, CW), lambda m, n: (n, 0, 0)),
        ],
        out_specs=pl.BlockSpec((MB, CW), lambda m, n: (n * (N // MB) + m, 0)),
        out_shape=jax.ShapeDtypeStruct((16 * N, CW), jnp.float32),
    )(x, wt, b16)


def _update_kernel(h_ref, a0, a1, a2, a3, s0, s1, s2, s3, g_ref, be_ref, o_ref):
    aggr = jnp.concatenate([a0[...], a1[...], a2[...], a3[...]], axis=1)
    sm = jnp.concatenate([s0[...], s1[...], s2[...], s3[...]], axis=1)
    bnv = (aggr + sm) * (BN_SCALE * g_ref[...][None, :]) + be_ref[...][None, :]
    o_ref[...] = h_ref[...] + jnp.maximum(bnv, 0.0)


def _update(h, aggr, yt, gam, bet):
    nblk = N // MB
    aspec = [pl.BlockSpec((MB, CW), (lambda m, g=g: (g * nblk + m, 0)))
             for g in range(NCHUNK)]
    sspec = [pl.BlockSpec((MB, CW), (lambda m, g=g: ((12 + g) * nblk + m, 0)))
             for g in range(NCHUNK)]
    return pl.pallas_call(
        _update_kernel,
        grid=(nblk,),
        in_specs=[pl.BlockSpec((MB, H), lambda m: (m, 0))] + aspec + sspec + [
            pl.BlockSpec((H,), lambda m: (0,)),
            pl.BlockSpec((H,), lambda m: (0,)),
        ],
        out_specs=pl.BlockSpec((MB, H), lambda m: (m, 0)),
        out_shape=jax.ShapeDtypeStruct((N, H), jnp.float32),
    )(h, aggr, aggr, aggr, aggr, yt, yt, yt, yt, gam, bet)


def _edge_body(yt, srcd, dstd, out,
               didx0, sidx0, kidx0, qidx0, vidx0, tbuf0, vbuf0, sdidx0,
               didx1, sidx1, kidx1, qidx1, vidx1, tbuf1, vbuf1, sdidx1,
               zbuf, acc,
               semi0, semk0, semq0, semv0, semsc0,
               semi1, semk1, semq1, semv1, semsc1):
    c = lax.axis_index("c")
    s = lax.axis_index("s")

    BUFS = [
        (didx0, sidx0, kidx0, qidx0, vidx0, tbuf0, vbuf0,
         semi0, semk0, semq0, semv0, sdidx0, semsc0),
        (didx1, sidx1, kidx1, qidx1, vidx1, tbuf1, vbuf1,
         semi1, semk1, semq1, semv1, sdidx1, semsc1),
    ]

    def zrow(r, _):
        for i in range(CW // 16):
            zbuf[r, pl.ds(i * 16, 16)] = jnp.zeros((16,), jnp.float32)
        return 0
    lax.fori_loop(0, ZRZ, zrow, 0)

    start = s * ROWS_A
    rows_mine = jnp.where(s < NS - 1, ROWS_A, N - (NS - 1) * ROWS_A)
    nzero = rows_mine // ZRZ
    ncopies = rows_mine // ZR

    def idx_load_async(b, p):
        didx, sidx = BUFS[p][0], BUFS[p][1]
        semi = BUFS[p][7]
        e0 = s * EPS + b * B
        pltpu.async_copy(dstd.at[pl.ds(e0, B)], didx, semi)
        pltpu.async_copy(srcd.at[pl.ds(e0, B)], sidx, semi)

    def idx_wait(b, p):
        didx, sidx = BUFS[p][0], BUFS[p][1]
        semi = BUFS[p][7]
        e0 = s * EPS + b * B
        pltpu.make_async_copy(dstd.at[pl.ds(e0, B)], didx, semi).wait()
        pltpu.make_async_copy(srcd.at[pl.ds(e0, B)], sidx, semi).wait()

    for half in range(2):
        g = c * 2 + half
        def zc(j, _):
            pltpu.sync_copy(zbuf, acc.at[pl.ds(start + j * ZRZ, ZRZ)])
            return 0
        lax.fori_loop(0, nzero, zc, 0)
        plsc.subcore_barrier()

        off_k = g * N
        off_q = (4 + g) * N
        off_v = (8 + g) * N

        def shift(p):
            didx, sidx, kidx, qidx, vidx = BUFS[p][:5]
            for i in range(B // 16):
                sl = pl.ds(i * 16, 16)
                d16 = didx[sl]
                s16 = sidx[sl]
                kidx[sl] = d16 + off_k
                qidx[sl] = s16 + off_q
                vidx[sl] = s16 + off_v

        def issue_kv(p):
            kidx, _, vidx, tbuf, vbuf = BUFS[p][2:7]
            semk, _, semv = BUFS[p][8:11]
            pltpu.async_copy(yt.at[kidx], tbuf, semk)
            pltpu.async_copy(yt.at[vidx], vbuf, semv)

        def step(b, p, first=False):
            didx = BUFS[p][0]
            kidx, qidx, vidx, tbuf, vbuf = BUFS[p][2:7]
            semk, semq, semv = BUFS[p][8:11]
            sdidx, semsc = BUFS[p][11:13]
            p1 = 1 - p
            tbuf1, vbuf1 = BUFS[p1][5:7]
            kidx1, qidx1 = BUFS[p1][2:4]
            semk1, semq1 = BUFS[p1][8:10]
            sdidx1, semsc1 = BUFS[p1][11:13]

            @pl.when(b + 1 < NB)
            def _():
                idx_wait(b + 1, p1)
                shift(p1)

            if not first:
                pltpu.make_async_copy(vbuf1, acc.at[sdidx1], semsc1).wait()

            @pl.when(b + 1 < NB)
            def _():
                issue_kv(p1)

            pltpu.make_async_copy(yt.at[qidx], tbuf, semq).wait()
            pltpu.make_async_copy(yt.at[vidx], vbuf, semv).wait()

            def edge(e, _):
                for i in range(CW // 16):
                    sl = pl.ds(i * 16, 16)
                    t = tbuf[e, sl]
                    vv = vbuf[e, sl]
                    vbuf[e, sl] = vv / (1.0 + jnp.exp(-t))
                return 0
            lax.fori_loop(0, B, edge, 0)

            for i in range(B // 16):
                sl = pl.ds(i * 16, 16)
                sdidx[sl] = didx[sl]
            pltpu.async_copy(vbuf, acc.at[sdidx], semsc, add=True)

            @pl.when(b + 2 < NB)
            def _():
                idx_load_async(b + 2, p)

            @pl.when(b + 1 < NB)
            def _():
                pltpu.make_async_copy(yt.at[kidx1], tbuf1, semk1).wait()
                pltpu.async_copy(yt.at[qidx1], tbuf1, semq1, add=True)

        idx_load_async(0, 0)
        idx_wait(0, 0)
        shift(0)
        issue_kv(0)
        pltpu.make_async_copy(yt.at[BUFS[0][2]], BUFS[0][5], BUFS[0][8]).wait()
        pltpu.async_copy(yt.at[BUFS[0][3]], BUFS[0][5], BUFS[0][9], add=True)
        idx_load_async(1, 1)
        step(0, 0, first=True)

        def pair(i, _):
            step(2 * i + 1, 1)
            step(2 * i + 2, 0)
            return 0
        lax.fori_loop(0, (NB - 1) // 2, pair, 0)

        pltpu.make_async_copy(BUFS[0][6], acc.at[BUFS[0][11]], BUFS[0][12]).wait()

        plsc.subcore_barrier()

        def fc(j, _):
            pltpu.sync_copy(acc.at[pl.ds(start + j * ZR, ZR)],
                            out.at[pl.ds(g * N + start + j * ZR, ZR)])
            return 0
        lax.fori_loop(0, ncopies, fc, 0)
        plsc.subcore_barrier()


_edge_call = pl.kernel(
    _edge_body,
    out_type=jax.ShapeDtypeStruct((NCHUNK * N, CW), jnp.float32),
    mesh=plsc.VectorSubcoreMesh(core_axis_name="c", subcore_axis_name="s",
                                num_cores=2, num_subcores=NS),
    scratch_types=(
        [pltpu.VMEM((B,), jnp.int32)] * 5
        + [pltpu.VMEM((B, CW), jnp.float32)] * 2
        + [pltpu.VMEM((B,), jnp.int32)]
        + [pltpu.VMEM((B,), jnp.int32)] * 5
        + [pltpu.VMEM((B, CW), jnp.float32)] * 2
        + [pltpu.VMEM((B,), jnp.int32)]
        + [pltpu.VMEM((ZRZ, CW), jnp.float32)]
        + [pltpu.VMEM_SHARED((N, CW), jnp.float32)]
        + [pltpu.SemaphoreType.DMA] * 10
    ),
)


def kernel(x, edge_index, Win, b_in, Wk, bk, Wq, bq, Wv, bv, Ws, bs,
           gamma, beta, Wpost, bpost):
    src = edge_index[0]
    dst = edge_index[1]
    L = Wk.shape[0]

    h = _mm(x, Win.T, b_in, relu=True)
    for l in range(L):
        wt = jnp.concatenate([Wk[l], Wq[l], Wv[l], Ws[l]], axis=0).T
        bf = jnp.concatenate([bk[l], bq[l], bv[l], bs[l]]).reshape(16, 1, CW)
        yt = _mmc(h, wt, bf)
        aggr = _edge_call(yt, src, dst)
        h = _update(h, aggr, yt, gamma[l], beta[l])
    return _mm(h, Wpost.T, bpost)


# speedup vs baseline: 4.0520x; 1.0106x over previous
import functools

import jax
import jax.numpy as jnp
from jax import lax
from jax.experimental import pallas as pl
from jax.experimental.pallas import tpu as pltpu
from jax.experimental.pallas import tpu_sc as plsc

N = 10000
H = 512
E = 160000
MB = 1000
CW = 128
NCHUNK = H // CW
NS = 16
B = 80
EPS = E // NS
NB = EPS // B
ROWS_A = 640
ZR = 80
ZRZ = 40
BN_SCALE = 1.0 / (1.0 + 1e-5) ** 0.5


def _mm_kernel(x_ref, wt_ref, b_ref, o_ref, *, relu):
    y = jnp.dot(x_ref[...].astype(jnp.bfloat16), wt_ref[...].astype(jnp.bfloat16),
                preferred_element_type=jnp.float32) + b_ref[...][None, :]
    if relu:
        y = jnp.maximum(y, 0.0)
    o_ref[...] = y


def _mm(x, wt, b, relu=False):
    n, k = x.shape
    m = wt.shape[1]
    return pl.pallas_call(
        functools.partial(_mm_kernel, relu=relu),
        grid=(n // MB,),
        in_specs=[
            pl.BlockSpec((MB, k), lambda i: (i, 0)),
            pl.BlockSpec((k, m), lambda i: (0, 0)),
            pl.BlockSpec((m,), lambda i: (0,)),
        ],
        out_specs=pl.BlockSpec((MB, m), lambda i: (i, 0)),
        out_shape=jax.ShapeDtypeStruct((n, m), jnp.float32),
    )(x, wt, b)


def _mmc_kernel(x_ref, wt_ref, b_ref, o_ref):
    o_ref[...] = jnp.dot(x_ref[...].astype(jnp.bfloat16),
                         wt_ref[...].astype(jnp.bfloat16),
                         preferred_element_type=jnp.float32) + b_ref[0, 0, :][None, :]


def _mmc(x, wt, b16):
    return pl.pallas_call(
        _mmc_kernel,
        grid=(N // MB, 16),
        in_specs=[
            pl.BlockSpec((MB, H), lambda m, n: (m, 0)),
            pl.BlockSpec((H, CW), lambda m, n: (0, n)),
            pl.BlockSpec((1, 1, CW), lambda m, n: (n, 0, 0)),
        ],
        out_specs=pl.BlockSpec((MB, CW), lambda m, n: (n * (N // MB) + m, 0)),
        out_shape=jax.ShapeDtypeStruct((16 * N, CW), jnp.float32),
    )(x, wt, b16)


def _update_kernel(h_ref, a0, a1, a2, a3, s0, s1, s2, s3, g_ref, be_ref, o_ref):
    aggr = jnp.concatenate([a0[...], a1[...], a2[...], a3[...]], axis=1)
    sm = jnp.concatenate([s0[...], s1[...], s2[...], s3[...]], axis=1)
    bnv = (aggr + sm) * (BN_SCALE * g_ref[...][None, :]) + be_ref[...][None, :]
    o_ref[...] = h_ref[...] + jnp.maximum(bnv, 0.0)


def _update(h, aggr, yt, gam, bet):
    nblk = N // MB
    aspec = [pl.BlockSpec((MB, CW), (lambda m, g=g: (g * nblk + m, 0)))
             for g in range(NCHUNK)]
    sspec = [pl.BlockSpec((MB, CW), (lambda m, g=g: ((12 + g) * nblk + m, 0)))
             for g in range(NCHUNK)]
    return pl.pallas_call(
        _update_kernel,
        grid=(nblk,),
        in_specs=[pl.BlockSpec((MB, H), lambda m: (m, 0))] + aspec + sspec + [
            pl.BlockSpec((H,), lambda m: (0,)),
            pl.BlockSpec((H,), lambda m: (0,)),
        ],
        out_specs=pl.BlockSpec((MB, H), lambda m: (m, 0)),
        out_shape=jax.ShapeDtypeStruct((N, H), jnp.float32),
    )(h, aggr, aggr, aggr, aggr, yt, yt, yt, yt, gam, bet)


def _fuse_in_kernel(x_ref, win_ref, bin_ref, wt_ref, b_ref, h_ref, yt_ref, hs):
    n = pl.program_id(1)

    @pl.when(n == 0)
    def _():
        h = jnp.dot(x_ref[...].astype(jnp.bfloat16),
                    win_ref[...].astype(jnp.bfloat16),
                    preferred_element_type=jnp.float32) + bin_ref[...][None, :]
        h = jnp.maximum(h, 0.0)
        hs[...] = h
        h_ref[...] = h

    yt_ref[...] = jnp.dot(hs[...].astype(jnp.bfloat16),
                          wt_ref[...].astype(jnp.bfloat16),
                          preferred_element_type=jnp.float32) + b_ref[0, 0, :][None, :]


def _fuse_in(x, winT, b_in, wt, b16):
    nblk = N // MB
    return pl.pallas_call(
        _fuse_in_kernel,
        grid=(nblk, 16),
        in_specs=[
            pl.BlockSpec((MB, x.shape[1]), lambda m, n: (m, 0)),
            pl.BlockSpec((x.shape[1], H), lambda m, n: (0, 0)),
            pl.BlockSpec((H,), lambda m, n: (0,)),
            pl.BlockSpec((H, CW), lambda m, n: (0, n)),
            pl.BlockSpec((1, 1, CW), lambda m, n: (n, 0, 0)),
        ],
        out_specs=[
            pl.BlockSpec((MB, H), lambda m, n: (m, 0)),
            pl.BlockSpec((MB, CW), lambda m, n: (n * nblk + m, 0)),
        ],
        out_shape=[
            jax.ShapeDtypeStruct((N, H), jnp.float32),
            jax.ShapeDtypeStruct((16 * N, CW), jnp.float32),
        ],
        scratch_shapes=[pltpu.VMEM((MB, H), jnp.float32)],
    )(x, winT, b_in, wt, b16)


def _fuse_mid_kernel(h_ref, a0, a1, a2, a3, s0, s1, s2, s3, g_ref, be_ref,
                     wt_ref, b_ref, h_out, yt_ref, hs):
    n = pl.program_id(1)

    @pl.when(n == 0)
    def _():
        aggr = jnp.concatenate([a0[...], a1[...], a2[...], a3[...]], axis=1)
        sm = jnp.concatenate([s0[...], s1[...], s2[...], s3[...]], axis=1)
        bnv = (aggr + sm) * (BN_SCALE * g_ref[...][None, :]) + be_ref[...][None, :]
        h = h_ref[...] + jnp.maximum(bnv, 0.0)
        hs[...] = h
        h_out[...] = h

    yt_ref[...] = jnp.dot(hs[...].astype(jnp.bfloat16),
                          wt_ref[...].astype(jnp.bfloat16),
                          preferred_element_type=jnp.float32) + b_ref[0, 0, :][None, :]


def _fuse_mid(h, aggr, yt, gam, bet, wt, b16):
    nblk = N // MB
    aspec = [pl.BlockSpec((MB, CW), (lambda m, n, g=g: (g * nblk + m, 0)))
             for g in range(NCHUNK)]
    sspec = [pl.BlockSpec((MB, CW), (lambda m, n, g=g: ((12 + g) * nblk + m, 0)))
             for g in range(NCHUNK)]
    return pl.pallas_call(
        _fuse_mid_kernel,
        grid=(nblk, 16),
        in_specs=[pl.BlockSpec((MB, H), lambda m, n: (m, 0))] + aspec + sspec + [
            pl.BlockSpec((H,), lambda m, n: (0,)),
            pl.BlockSpec((H,), lambda m, n: (0,)),
            pl.BlockSpec((H, CW), lambda m, n: (0, n)),
            pl.BlockSpec((1, 1, CW), lambda m, n: (n, 0, 0)),
        ],
        out_specs=[
            pl.BlockSpec((MB, H), lambda m, n: (m, 0)),
            pl.BlockSpec((MB, CW), lambda m, n: (n * nblk + m, 0)),
        ],
        out_shape=[
            jax.ShapeDtypeStruct((N, H), jnp.float32),
            jax.ShapeDtypeStruct((16 * N, CW), jnp.float32),
        ],
        scratch_shapes=[pltpu.VMEM((MB, H), jnp.float32)],
    )(h, aggr, aggr, aggr, aggr, yt, yt, yt, yt, gam, bet, wt, b16)


def _fuse_out_kernel(h_ref, a0, a1, a2, a3, s0, s1, s2, s3, g_ref, be_ref,
                     wt_ref, b_ref, o_ref):
    aggr = jnp.concatenate([a0[...], a1[...], a2[...], a3[...]], axis=1)
    sm = jnp.concatenate([s0[...], s1[...], s2[...], s3[...]], axis=1)
    bnv = (aggr + sm) * (BN_SCALE * g_ref[...][None, :]) + be_ref[...][None, :]
    h = h_ref[...] + jnp.maximum(bnv, 0.0)
    o_ref[...] = jnp.dot(h.astype(jnp.bfloat16), wt_ref[...].astype(jnp.bfloat16),
                         preferred_element_type=jnp.float32) + b_ref[...][None, :]


def _fuse_out(h, aggr, yt, gam, bet, wpostT, bpost):
    nblk = N // MB
    dout = wpostT.shape[1]
    aspec = [pl.BlockSpec((MB, CW), (lambda m, g=g: (g * nblk + m, 0)))
             for g in range(NCHUNK)]
    sspec = [pl.BlockSpec((MB, CW), (lambda m, g=g: ((12 + g) * nblk + m, 0)))
             for g in range(NCHUNK)]
    return pl.pallas_call(
        _fuse_out_kernel,
        grid=(nblk,),
        in_specs=[pl.BlockSpec((MB, H), lambda m: (m, 0))] + aspec + sspec + [
            pl.BlockSpec((H,), lambda m: (0,)),
            pl.BlockSpec((H,), lambda m: (0,)),
            pl.BlockSpec((H, dout), lambda m: (0, 0)),
            pl.BlockSpec((dout,), lambda m: (0,)),
        ],
        out_specs=pl.BlockSpec((MB, dout), lambda m: (m, 0)),
        out_shape=jax.ShapeDtypeStruct((N, dout), jnp.float32),
    )(h, aggr, aggr, aggr, aggr, yt, yt, yt, yt, gam, bet, wpostT, bpost)


def _edge_body(yt, srcd, dstd, out,
               didx0, sidx0, kidx0, qidx0, vidx0, tbuf0, vbuf0, sdidx0,
               didx1, sidx1, kidx1, qidx1, vidx1, tbuf1, vbuf1, sdidx1,
               zbuf, acc,
               semi0, semk0, semq0, semv0, semsc0,
               semi1, semk1, semq1, semv1, semsc1):
    c = lax.axis_index("c")
    s = lax.axis_index("s")

    BUFS = [
        (didx0, sidx0, kidx0, qidx0, vidx0, tbuf0, vbuf0,
         semi0, semk0, semq0, semv0, sdidx0, semsc0),
        (didx1, sidx1, kidx1, qidx1, vidx1, tbuf1, vbuf1,
         semi1, semk1, semq1, semv1, sdidx1, semsc1),
    ]

    def zrow(r, _):
        for i in range(CW // 16):
            zbuf[r, pl.ds(i * 16, 16)] = jnp.zeros((16,), jnp.float32)
        return 0
    lax.fori_loop(0, ZRZ, zrow, 0)

    start = s * ROWS_A
    rows_mine = jnp.where(s < NS - 1, ROWS_A, N - (NS - 1) * ROWS_A)
    nzero = rows_mine // ZRZ
    ncopies = rows_mine // ZR

    def idx_load_async(b, p):
        didx, sidx = BUFS[p][0], BUFS[p][1]
        semi = BUFS[p][7]
        e0 = s * EPS + b * B
        pltpu.async_copy(dstd.at[pl.ds(e0, B)], didx, semi)
        pltpu.async_copy(srcd.at[pl.ds(e0, B)], sidx, semi)

    def idx_wait(b, p):
        didx, sidx = BUFS[p][0], BUFS[p][1]
        semi = BUFS[p][7]
        e0 = s * EPS + b * B
        pltpu.make_async_copy(dstd.at[pl.ds(e0, B)], didx, semi).wait()
        pltpu.make_async_copy(srcd.at[pl.ds(e0, B)], sidx, semi).wait()

    for half in range(2):
        g = c * 2 + half
        def zc(j, _):
            pltpu.sync_copy(zbuf, acc.at[pl.ds(start + j * ZRZ, ZRZ)])
            return 0
        lax.fori_loop(0, nzero, zc, 0)
        plsc.subcore_barrier()

        off_k = g * N
        off_q = (4 + g) * N
        off_v = (8 + g) * N

        def shift(p):
            didx, sidx, kidx, qidx, vidx = BUFS[p][:5]
            for i in range(B // 16):
                sl = pl.ds(i * 16, 16)
                d16 = didx[sl]
                s16 = sidx[sl]
                kidx[sl] = d16 + off_k
                qidx[sl] = s16 + off_q
                vidx[sl] = s16 + off_v

        def issue_kv(p):
            kidx, _, vidx, tbuf, vbuf = BUFS[p][2:7]
            semk, _, semv = BUFS[p][8:11]
            pltpu.async_copy(yt.at[kidx], tbuf, semk)
            pltpu.async_copy(yt.at[vidx], vbuf, semv)

        def step(b, p, first=False):
            didx = BUFS[p][0]
            kidx, qidx, vidx, tbuf, vbuf = BUFS[p][2:7]
            semk, semq, semv = BUFS[p][8:11]
            sdidx, semsc = BUFS[p][11:13]
            p1 = 1 - p
            tbuf1, vbuf1 = BUFS[p1][5:7]
            kidx1, qidx1 = BUFS[p1][2:4]
            semk1, semq1 = BUFS[p1][8:10]
            sdidx1, semsc1 = BUFS[p1][11:13]

            @pl.when(b + 1 < NB)
            def _():
                idx_wait(b + 1, p1)
                shift(p1)

            if not first:
                pltpu.make_async_copy(vbuf1, acc.at[sdidx1], semsc1).wait()

            @pl.when(b + 1 < NB)
            def _():
                issue_kv(p1)

            pltpu.make_async_copy(yt.at[qidx], tbuf, semq).wait()
            pltpu.make_async_copy(yt.at[vidx], vbuf, semv).wait()

            def edge(e, _):
                for i in range(CW // 16):
                    sl = pl.ds(i * 16, 16)
                    t = tbuf[e, sl]
                    vv = vbuf[e, sl]
                    vbuf[e, sl] = vv / (1.0 + jnp.exp(-t))
                return 0
            lax.fori_loop(0, B, edge, 0)

            for i in range(B // 16):
                sl = pl.ds(i * 16, 16)
                sdidx[sl] = didx[sl]
            pltpu.async_copy(vbuf, acc.at[sdidx], semsc, add=True)

            @pl.when(b + 2 < NB)
            def _():
                idx_load_async(b + 2, p)

            @pl.when(b + 1 < NB)
            def _():
                pltpu.make_async_copy(yt.at[kidx1], tbuf1, semk1).wait()
                pltpu.async_copy(yt.at[qidx1], tbuf1, semq1, add=True)

        idx_load_async(0, 0)
        idx_wait(0, 0)
        shift(0)
        issue_kv(0)
        pltpu.make_async_copy(yt.at[BUFS[0][2]], BUFS[0][5], BUFS[0][8]).wait()
        pltpu.async_copy(yt.at[BUFS[0][3]], BUFS[0][5], BUFS[0][9], add=True)
        idx_load_async(1, 1)
        step(0, 0, first=True)

        def pair(i, _):
            step(2 * i + 1, 1)
            step(2 * i + 2, 0)
            return 0
        lax.fori_loop(0, (NB - 1) // 2, pair, 0)

        pltpu.make_async_copy(BUFS[0][6], acc.at[BUFS[0][11]], BUFS[0][12]).wait()

        plsc.subcore_barrier()

        def fc(j, _):
            pltpu.sync_copy(acc.at[pl.ds(start + j * ZR, ZR)],
                            out.at[pl.ds(g * N + start + j * ZR, ZR)])
            return 0
        lax.fori_loop(0, ncopies, fc, 0)
        plsc.subcore_barrier()


_edge_call = pl.kernel(
    _edge_body,
    out_type=jax.ShapeDtypeStruct((NCHUNK * N, CW), jnp.float32),
    mesh=plsc.VectorSubcoreMesh(core_axis_name="c", subcore_axis_name="s",
                                num_cores=2, num_subcores=NS),
    scratch_types=(
        [pltpu.VMEM((B,), jnp.int32)] * 5
        + [pltpu.VMEM((B, CW), jnp.float32)] * 2
        + [pltpu.VMEM((B,), jnp.int32)]
        + [pltpu.VMEM((B,), jnp.int32)] * 5
        + [pltpu.VMEM((B, CW), jnp.float32)] * 2
        + [pltpu.VMEM((B,), jnp.int32)]
        + [pltpu.VMEM((ZRZ, CW), jnp.float32)]
        + [pltpu.VMEM_SHARED((N, CW), jnp.float32)]
        + [pltpu.SemaphoreType.DMA] * 10
    ),
)


def kernel(x, edge_index, Win, b_in, Wk, bk, Wq, bq, Wv, bv, Ws, bs,
           gamma, beta, Wpost, bpost):
    src = edge_index[0]
    dst = edge_index[1]
    L = Wk.shape[0]

    wts = []
    bfs = []
    for l in range(L):
        wts.append(jnp.concatenate([Wk[l], Wq[l], Wv[l], Ws[l]], axis=0).T)
        bfs.append(jnp.concatenate([bk[l], bq[l], bv[l], bs[l]]).reshape(16, 1, CW))

    h, yt = _fuse_in(x, Win.T, b_in, wts[0], bfs[0])
    for l in range(L - 1):
        aggr = _edge_call(yt, src, dst)
        h, yt = _fuse_mid(h, aggr, yt, gamma[l], beta[l], wts[l + 1], bfs[l + 1])
    aggr = _edge_call(yt, src, dst)
    return _fuse_out(h, aggr, yt, gamma[L - 1], beta[L - 1], Wpost.T, bpost)
